# baseline (speedup 1.0000x reference)
import functools

import jax
import jax.numpy as jnp
from jax import lax
from jax.experimental import pallas as pl
from jax.experimental.pallas import tpu as pltpu
from jax.experimental.pallas import tpu_sc as plsc

_NB = 320000
_NT = 6
_H = 128
_NCOMB = _NT * _NT * _NT
_NW = 32
_CH = 128
_NCH = 78
_NBW = _NCH * _CH
_NEXTRA = (_NB - _NW * _NBW) // _CH


def _build_combined(e0_ref, e1_ref, e2_ref, c_ref):
    e2 = e2_ref[...]
    for i0 in range(_NT):
        r0 = e0_ref[i0, :][None, :]
        for i1 in range(_NT):
            r1 = e1_ref[i1, :][None, :]
            c_ref[pl.ds((i0 * _NT + i1) * _NT, _NT), :] = e2 + r0 + r1


@functools.partial(
    pl.kernel,
    out_type=jax.ShapeDtypeStruct((_NB, _H), jnp.float32),
    mesh=plsc.VectorSubcoreMesh(core_axis_name="c", subcore_axis_name="s"),
    scratch_types=[
        pltpu.VMEM_SHARED((_NCOMB, _H), jnp.float32),
        pltpu.VMEM((_NBW,), jnp.int32),
        pltpu.VMEM((_NBW,), jnp.int32),
        pltpu.VMEM((_NBW,), jnp.int32),
        pltpu.VMEM((_NBW,), jnp.int32),
        pltpu.VMEM((_CH,), jnp.int32),
        pltpu.VMEM((_CH, _H), jnp.float32),
        pltpu.VMEM((_CH, _H), jnp.float32),
        pltpu.VMEM((_CH, _H), jnp.float32),
        pltpu.VMEM((_CH, _H), jnp.float32),
        pltpu.SemaphoreType.DMA,
        pltpu.SemaphoreType.DMA,
        pltpu.SemaphoreType.DMA,
        pltpu.SemaphoreType.DMA,
        pltpu.SemaphoreType.DMA,
        pltpu.SemaphoreType.DMA,
        pltpu.SemaphoreType.DMA,
        pltpu.SemaphoreType.DMA,
    ],
)
def _sc_encode(c_hbm, a0_hbm, a1_hbm, a2_hbm, out_hbm,
               c_sp, a0_v, a1_v, a2_v, idx_v, idx_x,
               buf0, buf1, buf2, buf3,
               sg0, sg1, sg2, sg3, ss0, ss1, ss2, ss3):
    sub = lax.axis_index("s")
    wid = sub * 2 + lax.axis_index("c")
    row0 = wid * _NBW
    bufs = (buf0, buf1, buf2, buf3)
    sgs = (sg0, sg1, sg2, sg3)
    sss = (ss0, ss1, ss2, ss3)

    @pl.when(sub == 0)
    def _():
        pltpu.sync_copy(c_hbm, c_sp)

    pltpu.sync_copy(a0_hbm.at[pl.ds(row0, _NBW)], a0_v)
    pltpu.sync_copy(a1_hbm.at[pl.ds(row0, _NBW)], a1_v)
    pltpu.sync_copy(a2_hbm.at[pl.ds(row0, _NBW)], a2_v)

    def fuse_grp(g, carry):
        sl = pl.ds(g * 16, 16)
        idx_v[sl] = (a0_v[sl] * _NT + a1_v[sl]) * _NT + a2_v[sl]
        return carry

    lax.fori_loop(0, _NBW // 16, fuse_grp, 0)

    plsc.subcore_barrier()

    def gstart(c, b):
        pltpu.async_copy(
            c_sp.at[idx_v.at[pl.ds(c * _CH, _CH)]], bufs[b], sgs[b])

    def gwait(c, b):
        pltpu.make_async_copy(
            c_sp.at[idx_v.at[pl.ds(c * _CH, _CH)]], bufs[b], sgs[b]).wait()

    def sstart(c, b):
        pltpu.async_copy(
            bufs[b], out_hbm.at[pl.ds(row0 + c * _CH, _CH)], sss[b])

    def swait(c, b):
        pltpu.make_async_copy(
            bufs[b], out_hbm.at[pl.ds(row0 + c * _CH, _CH)], sss[b]).wait()

    gstart(0, 0)
    gstart(1, 1)

    def quad_body(g, carry):
        for d in range(4):
            c = g * 4 + d
            @pl.when(c >= 2)
            def _():
                swait(c - 2, (d + 2) % 4)

            gstart(c + 2, (d + 2) % 4)
            gwait(c, d)
            sstart(c, d)
        return carry

    lax.fori_loop(0, (_NCH - 2) // 4, quad_body, 0)

    swait(74, 2)
    gwait(76, 0)
    sstart(76, 0)
    swait(75, 3)
    gwait(77, 1)
    sstart(77, 1)
    swait(76, 0)
    swait(77, 1)

    @pl.when(wid < _NEXTRA)
    def _():
        xrow = _NW * _NBW + wid * _CH
        pltpu.sync_copy(a0_hbm.at[pl.ds(xrow, _CH)], a0_v.at[pl.ds(0, _CH)])
        pltpu.sync_copy(a1_hbm.at[pl.ds(xrow, _CH)], a1_v.at[pl.ds(0, _CH)])
        pltpu.sync_copy(a2_hbm.at[pl.ds(xrow, _CH)], a2_v.at[pl.ds(0, _CH)])
        for k in range(_CH // 16):
            sl = pl.ds(k * 16, 16)
            idx_x[sl] = (a0_v[sl] * _NT + a1_v[sl]) * _NT + a2_v[sl]
        pltpu.async_copy(c_sp.at[idx_x], buf0, sg0).wait()
        pltpu.sync_copy(buf0, out_hbm.at[pl.ds(xrow, _CH)])


def kernel(bond_attr, emb0, emb1, emb2):
    comb = pl.pallas_call(
        _build_combined,
        out_shape=jax.ShapeDtypeStruct((_NCOMB, _H), jnp.float32),
    )(emb0, emb1, emb2)

    a = bond_attr.astype(jnp.int32)
    return _sc_encode(comb, a[:, 0], a[:, 1], a[:, 2])


# speedup vs baseline: 1.0377x; 1.0377x over previous
import functools

import jax
import jax.numpy as jnp
from jax import lax
from jax.experimental import pallas as pl
from jax.experimental.pallas import tpu as pltpu
from jax.experimental.pallas import tpu_sc as plsc

_NB = 320000
_NT = 6
_H = 128
_NCOMB = _NT * _NT * _NT
_NW = 32
_CH = 128
_NCH = 78
_NEXTRA = (_NB - _NW * _NCH * _CH) // _CH


def _tc_prelude(a0_ref, a1_ref, a2_ref, e0_ref, e1_ref, e2_ref,
                c_ref, idx_ref):
    e2 = e2_ref[...]
    for i0 in range(_NT):
        r0 = e0_ref[i0, :][None, :]
        for i1 in range(_NT):
            r1 = e1_ref[i1, :][None, :]
            c_ref[pl.ds((i0 * _NT + i1) * _NT, _NT), :] = e2 + r0 + r1

    idx_ref[...] = (a0_ref[...] * _NT + a1_ref[...]) * _NT + a2_ref[...]


@functools.partial(
    pl.kernel,
    out_type=jax.ShapeDtypeStruct((_NB, _H), jnp.float32),
    mesh=plsc.VectorSubcoreMesh(core_axis_name="c", subcore_axis_name="s"),
    compiler_params=pltpu.CompilerParams(needs_layout_passes=False),
    scratch_types=[
        pltpu.VMEM_SHARED((_NCOMB, _H), jnp.float32),
        pltpu.VMEM((_NCH * _CH,), jnp.int32),
        pltpu.VMEM((_CH,), jnp.int32),
        pltpu.VMEM((_CH, _H), jnp.float32),
        pltpu.VMEM((_CH, _H), jnp.float32),
        pltpu.VMEM((_CH, _H), jnp.float32),
        pltpu.VMEM((_CH, _H), jnp.float32),
        pltpu.SemaphoreType.DMA,
        pltpu.SemaphoreType.DMA,
        pltpu.SemaphoreType.DMA,
        pltpu.SemaphoreType.DMA,
        pltpu.SemaphoreType.DMA,
        pltpu.SemaphoreType.DMA,
        pltpu.SemaphoreType.DMA,
        pltpu.SemaphoreType.DMA,
    ],
)
def _sc_gather(c_hbm, idx_hbm, out_hbm, c_sp, idx_v, idx_x,
               buf0, buf1, buf2, buf3,
               sg0, sg1, sg2, sg3, ss0, ss1, ss2, ss3):
    sub = lax.axis_index("s")
    wid = sub * 2 + lax.axis_index("c")
    row0 = wid * (_NCH * _CH)
    bufs = (buf0, buf1, buf2, buf3)
    sgs = (sg0, sg1, sg2, sg3)
    sss = (ss0, ss1, ss2, ss3)

    @pl.when(sub == 0)
    def _():
        pltpu.sync_copy(c_hbm, c_sp)

    pltpu.sync_copy(idx_hbm.at[pl.ds(row0, _NCH * _CH)], idx_v)

    plsc.subcore_barrier()

    def gstart(c, b):
        pltpu.async_copy(
            c_sp.at[idx_v.at[pl.ds(c * _CH, _CH)]], bufs[b], sgs[b])

    def gwait(c, b):
        pltpu.make_async_copy(
            c_sp.at[idx_v.at[pl.ds(c * _CH, _CH)]], bufs[b], sgs[b]).wait()

    def sstart(c, b):
        pltpu.async_copy(
            bufs[b], out_hbm.at[pl.ds(row0 + c * _CH, _CH)], sss[b])

    def swait(c, b):
        pltpu.make_async_copy(
            bufs[b], out_hbm.at[pl.ds(row0 + c * _CH, _CH)], sss[b]).wait()

    gstart(0, 0)
    gstart(1, 1)

    def quad_body(g, carry):
        for d in range(4):
            c = g * 4 + d
            @pl.when(c >= 2)
            def _():
                swait(c - 2, (d + 2) % 4)

            gstart(c + 2, (d + 2) % 4)
            gwait(c, d)
            sstart(c, d)
        return carry

    lax.fori_loop(0, (_NCH - 2) // 4, quad_body, 0)

    swait(74, 2)
    gwait(76, 0)
    sstart(76, 0)
    swait(75, 3)
    gwait(77, 1)
    sstart(77, 1)
    swait(76, 0)
    swait(77, 1)

    @pl.when(wid < _NEXTRA)
    def _():
        xrow = _NW * _NCH * _CH + wid * _CH
        pltpu.sync_copy(idx_hbm.at[pl.ds(xrow, _CH)], idx_x)
        pltpu.async_copy(c_sp.at[idx_x], buf0, sg0).wait()
        pltpu.sync_copy(buf0, out_hbm.at[pl.ds(xrow, _CH)])


def kernel(bond_attr, emb0, emb1, emb2):
    a = bond_attr.astype(jnp.int32)
    a0 = a[:, 0].reshape(_NB // _H, _H)
    a1 = a[:, 1].reshape(_NB // _H, _H)
    a2 = a[:, 2].reshape(_NB // _H, _H)

    comb, idx = pl.pallas_call(
        _tc_prelude,
        out_shape=(
            jax.ShapeDtypeStruct((_NCOMB, _H), jnp.float32),
            jax.ShapeDtypeStruct((_NB // _H, _H), jnp.int32),
        ),
    )(a0, a1, a2, emb0, emb1, emb2)

    return _sc_gather(comb, idx.reshape(_NB))
